# baseline (speedup 1.0000x reference)
import functools

import jax
import jax.numpy as jnp
from jax import lax
from jax.experimental import pallas as pl
from jax.experimental.pallas import tpu as pltpu
from jax.experimental.pallas import tpu_sc as plsc

L = 16
IDX_W = 128


def _make_composer(B, N, D):
    info = plsc.get_sparse_core_info()
    nc, ns = info.num_cores, info.num_subcores
    nw = nc * ns
    chunk = B // nw
    n_gather = chunk // IDX_W
    W = D + D + 2

    mesh = plsc.VectorSubcoreMesh(core_axis_name="c", subcore_axis_name="s")

    @functools.partial(
        pl.kernel,
        mesh=mesh,
        compiler_params=pltpu.CompilerParams(use_tc_tiling_on_sc=False,
                                             needs_layout_passes=False),
        out_type=jax.ShapeDtypeStruct((B, W), jnp.float32),
        scratch_types=[
            pltpu.VMEM((chunk, 3), jnp.int32),
            pltpu.VMEM((n_gather, IDX_W), jnp.int32),
            pltpu.VMEM((n_gather, IDX_W), jnp.int32),
            pltpu.VMEM((2, IDX_W, D), jnp.float32),
            pltpu.VMEM((2, IDX_W, D), jnp.float32),
            pltpu.VMEM((chunk, W), jnp.float32),
            pltpu.SemaphoreType.DMA,
            pltpu.SemaphoreType.DMA,
            pltpu.SemaphoreType.DMA,
            pltpu.SemaphoreType.DMA,
        ],
    )
    def composer(batch_hbm, s_hbm, out_hbm, batch_v, idx_obj, idx_nb,
                 st_obj, st_nb, rows_v, sem_o0, sem_o1, sem_n0, sem_n1):
        sem_o = (sem_o0, sem_o1)
        sem_n = (sem_n0, sem_n1)
        cid = lax.axis_index("c")
        sid = lax.axis_index("s")
        wid = cid * ns + sid
        base = wid * chunk

        pltpu.sync_copy(batch_hbm.at[pl.ds(base, chunk)], batch_v)

        iota = lax.iota(jnp.int32, L)
        c0 = jnp.zeros((L,), jnp.int32)
        c1 = jnp.full((L,), 1, jnp.int32)
        c2 = jnp.full((L,), 2, jnp.int32)
        ct = jnp.full((L,), 2 * D, jnp.int32)
        co = jnp.full((L,), 2 * D + 1, jnp.int32)

        for j in range(chunk // L):
            r = iota + j * L
            o = plsc.load_gather(batch_v, [r, c0])
            n = plsc.load_gather(batch_v, [r, c1])
            t = plsc.load_gather(batch_v, [r, c2])
            idx_obj[j // (IDX_W // L), pl.ds((j % (IDX_W // L)) * L, L)] = o
            idx_nb[j // (IDX_W // L), pl.ds((j % (IDX_W // L)) * L, L)] = n
            plsc.store_scatter(rows_v, [r, ct], t.astype(jnp.float32))
            plsc.store_scatter(rows_v, [r, co], o.astype(jnp.float32))

        def fire(g):
            ho = pltpu.async_copy(s_hbm.at[idx_obj.at[g]],
                                  st_obj.at[g % 2], sem_o[g % 2])
            hn = pltpu.async_copy(s_hbm.at[idx_nb.at[g]],
                                  st_nb.at[g % 2], sem_n[g % 2])
            return ho, hn

        pend = fire(0)
        for g in range(n_gather):
            ho, hn = pend
            if g + 1 < n_gather:
                nxt = fire(g + 1)
            ho.wait()
            hn.wait()
            b = g % 2
            grow = g * IDX_W

            def body(r, carry):
                for c in range(D // L):
                    rows_v[grow + r, pl.ds(c * L, L)] = (
                        st_obj[b, r, pl.ds(c * L, L)])
                for c in range(D // L):
                    rows_v[grow + r, pl.ds(D + c * L, L)] = (
                        st_nb[b, r, pl.ds(c * L, L)])
                return carry

            lax.fori_loop(0, IDX_W, body, 0)
            if g + 1 < n_gather:
                pend = nxt

        pltpu.sync_copy(rows_v, out_hbm.at[pl.ds(base, chunk)])

    return composer


def kernel(batch, S):
    B = batch.shape[0]
    N, D = S.shape
    return _make_composer(B, N, D)(batch, S)


# speedup vs baseline: 1.1040x; 1.1040x over previous
import functools

import jax
import jax.numpy as jnp
from jax import lax
from jax.experimental import pallas as pl
from jax.experimental.pallas import tpu as pltpu
from jax.experimental.pallas import tpu_sc as plsc

L = 16
IDX_W = 64
PW = 128


def _make_composer(B, N, D):
    info = plsc.get_sparse_core_info()
    nc, ns = info.num_cores, info.num_subcores
    nw = nc * ns
    chunk = B // nw
    n_gather = chunk // IDX_W
    W = D + D + 2

    mesh = plsc.VectorSubcoreMesh(core_axis_name="c", subcore_axis_name="s")

    @functools.partial(
        pl.kernel,
        mesh=mesh,
        compiler_params=pltpu.CompilerParams(use_tc_tiling_on_sc=False,
                                             needs_layout_passes=False),
        out_type=jax.ShapeDtypeStruct((B, W), jnp.float32),
        scratch_types=[
            pltpu.VMEM((chunk, 3), jnp.int32),
            pltpu.VMEM((n_gather, IDX_W), jnp.int32),
            pltpu.VMEM((n_gather, IDX_W), jnp.int32),
            pltpu.VMEM((2, IDX_W, PW), jnp.float32),
            pltpu.VMEM((2, IDX_W, PW), jnp.float32),
            pltpu.VMEM((chunk, W), jnp.float32),
            pltpu.SemaphoreType.DMA,
            pltpu.SemaphoreType.DMA,
            pltpu.SemaphoreType.DMA,
            pltpu.SemaphoreType.DMA,
        ],
    )
    def composer(batch_hbm, s_hbm, out_hbm, batch_v, idx_obj, idx_nb,
                 st_obj, st_nb, rows_v, sem_o0, sem_o1, sem_n0, sem_n1):
        sem_o = (sem_o0, sem_o1)
        sem_n = (sem_n0, sem_n1)
        cid = lax.axis_index("c")
        sid = lax.axis_index("s")
        wid = cid * ns + sid
        base = wid * chunk

        pltpu.sync_copy(batch_hbm.at[pl.ds(base, chunk)], batch_v)

        iota = lax.iota(jnp.int32, L)
        c0 = jnp.zeros((L,), jnp.int32)
        c1 = jnp.full((L,), 1, jnp.int32)
        c2 = jnp.full((L,), 2, jnp.int32)
        ct = jnp.full((L,), 2 * D, jnp.int32)
        co = jnp.full((L,), 2 * D + 1, jnp.int32)

        for j in range(chunk // L):
            r = iota + j * L
            o = plsc.load_gather(batch_v, [r, c0])
            n = plsc.load_gather(batch_v, [r, c1])
            t = plsc.load_gather(batch_v, [r, c2])
            idx_obj[j // (IDX_W // L), pl.ds((j % (IDX_W // L)) * L, L)] = o
            idx_nb[j // (IDX_W // L), pl.ds((j % (IDX_W // L)) * L, L)] = n
            plsc.store_scatter(rows_v, [r, ct], t.astype(jnp.float32))
            plsc.store_scatter(rows_v, [r, co], o.astype(jnp.float32))

        def fire(g):
            ho = pltpu.async_copy(s_hbm.at[idx_obj.at[g]],
                                  st_obj.at[g % 2], sem_o[g % 2])
            hn = pltpu.async_copy(s_hbm.at[idx_nb.at[g]],
                                  st_nb.at[g % 2], sem_n[g % 2])
            return ho, hn

        pend = fire(0)
        for g in range(n_gather):
            ho, hn = pend
            if g + 1 < n_gather:
                nxt = fire(g + 1)
            ho.wait()
            hn.wait()
            b = g % 2
            grow = g * IDX_W

            def body(r, carry):
                for c in range(D // L):
                    rows_v[grow + r, pl.ds(c * L, L)] = (
                        st_obj[b, r, pl.ds(c * L, L)])
                for c in range(D // L):
                    rows_v[grow + r, pl.ds(D + c * L, L)] = (
                        st_nb[b, r, pl.ds(c * L, L)])
                return carry

            lax.fori_loop(0, IDX_W, body, 0)
            if g + 1 < n_gather:
                pend = nxt

        pltpu.sync_copy(rows_v, out_hbm.at[pl.ds(base, chunk)])

    return composer


def kernel(batch, S):
    B = batch.shape[0]
    N, D = S.shape
    s_padded = jnp.pad(S, ((0, 0), (0, PW - D)))
    return _make_composer(B, N, D)(batch, s_padded)


# speedup vs baseline: 1.2193x; 1.1044x over previous
import functools

import jax
import jax.numpy as jnp
from jax import lax
from jax.experimental import pallas as pl
from jax.experimental.pallas import tpu as pltpu
from jax.experimental.pallas import tpu_sc as plsc

L = 16
IDX_W = 64
PW = 128


def _make_transpose_pad(N, D):
    BI = 2048
    grid = (N + BI - 1) // BI

    def body(s_t_ref, out_ref):
        out_ref[:, 0:D] = s_t_ref[...].T

    return pl.pallas_call(
        body,
        grid=(grid,),
        in_specs=[pl.BlockSpec((D, BI), lambda j: (0, j))],
        out_specs=pl.BlockSpec((BI, PW), lambda j: (j, 0)),
        out_shape=jax.ShapeDtypeStruct((N, PW), jnp.float32),
    )


def _make_composer(B, N, D):
    info = plsc.get_sparse_core_info()
    nc, ns = info.num_cores, info.num_subcores
    nw = nc * ns
    chunk = B // nw
    n_gather = chunk // IDX_W
    W = D + D + 2

    mesh = plsc.VectorSubcoreMesh(core_axis_name="c", subcore_axis_name="s")

    @functools.partial(
        pl.kernel,
        mesh=mesh,
        compiler_params=pltpu.CompilerParams(use_tc_tiling_on_sc=False,
                                             needs_layout_passes=False),
        out_type=jax.ShapeDtypeStruct((B, W), jnp.float32),
        scratch_types=[
            pltpu.VMEM((chunk, 3), jnp.int32),
            pltpu.VMEM((n_gather, IDX_W), jnp.int32),
            pltpu.VMEM((n_gather, IDX_W), jnp.int32),
            pltpu.VMEM((2, IDX_W, PW), jnp.float32),
            pltpu.VMEM((2, IDX_W, PW), jnp.float32),
            pltpu.VMEM((chunk, W), jnp.float32),
            pltpu.SemaphoreType.DMA,
            pltpu.SemaphoreType.DMA,
            pltpu.SemaphoreType.DMA,
            pltpu.SemaphoreType.DMA,
        ],
    )
    def composer(batch_hbm, s_hbm, out_hbm, batch_v, idx_obj, idx_nb,
                 st_obj, st_nb, rows_v, sem_o0, sem_o1, sem_n0, sem_n1):
        sem_o = (sem_o0, sem_o1)
        sem_n = (sem_n0, sem_n1)
        cid = lax.axis_index("c")
        sid = lax.axis_index("s")
        wid = cid * ns + sid
        base = wid * chunk

        pltpu.sync_copy(batch_hbm.at[pl.ds(base, chunk)], batch_v)

        iota = lax.iota(jnp.int32, L)
        c0 = jnp.zeros((L,), jnp.int32)
        c1 = jnp.full((L,), 1, jnp.int32)
        c2 = jnp.full((L,), 2, jnp.int32)
        ct = jnp.full((L,), 2 * D, jnp.int32)
        co = jnp.full((L,), 2 * D + 1, jnp.int32)

        for j in range(chunk // L):
            r = iota + j * L
            o = plsc.load_gather(batch_v, [r, c0])
            n = plsc.load_gather(batch_v, [r, c1])
            t = plsc.load_gather(batch_v, [r, c2])
            idx_obj[j // (IDX_W // L), pl.ds((j % (IDX_W // L)) * L, L)] = o
            idx_nb[j // (IDX_W // L), pl.ds((j % (IDX_W // L)) * L, L)] = n
            plsc.store_scatter(rows_v, [r, ct], t.astype(jnp.float32))
            plsc.store_scatter(rows_v, [r, co], o.astype(jnp.float32))

        def fire(g):
            ho = pltpu.async_copy(s_hbm.at[idx_obj.at[g]],
                                  st_obj.at[g % 2], sem_o[g % 2])
            hn = pltpu.async_copy(s_hbm.at[idx_nb.at[g]],
                                  st_nb.at[g % 2], sem_n[g % 2])
            return ho, hn

        pend = fire(0)
        for g in range(n_gather):
            ho, hn = pend
            if g + 1 < n_gather:
                nxt = fire(g + 1)
            ho.wait()
            hn.wait()
            b = g % 2
            grow = g * IDX_W

            def body(r, carry):
                for c in range(D // L):
                    rows_v[grow + r, pl.ds(c * L, L)] = (
                        st_obj[b, r, pl.ds(c * L, L)])
                for c in range(D // L):
                    rows_v[grow + r, pl.ds(D + c * L, L)] = (
                        st_nb[b, r, pl.ds(c * L, L)])
                return carry

            lax.fori_loop(0, IDX_W, body, 0)
            if g + 1 < n_gather:
                pend = nxt

        pltpu.sync_copy(rows_v, out_hbm.at[pl.ds(base, chunk)])

    return composer


def kernel(batch, S):
    B = batch.shape[0]
    N, D = S.shape
    s_padded = _make_transpose_pad(N, D)(S.T)
    return _make_composer(B, N, D)(batch, s_padded)


# speedup vs baseline: 1.2362x; 1.0138x over previous
import functools

import jax
import jax.numpy as jnp
from jax import lax
from jax.experimental import pallas as pl
from jax.experimental.pallas import tpu as pltpu
from jax.experimental.pallas import tpu_sc as plsc

L = 16
IDX_W = 64
PW = 128


def _make_transpose_pad(N, D):
    BI = 2048
    grid = (N + BI - 1) // BI

    def body(s_t_ref, out_ref):
        x = s_t_ref[...]
        eye = jnp.eye(128, dtype=jnp.float32)
        for c in range(BI // 128):
            xc = x[:, c * 128:(c + 1) * 128]
            xt = jax.lax.dot_general(
                eye, xc, dimension_numbers=(((1,), (1,)), ((), ())),
                preferred_element_type=jnp.float32)
            out_ref[pl.ds(c * 128, 128), 0:D] = xt

    return pl.pallas_call(
        body,
        grid=(grid,),
        in_specs=[pl.BlockSpec((D, BI), lambda j: (0, j))],
        out_specs=pl.BlockSpec((BI, PW), lambda j: (j, 0)),
        out_shape=jax.ShapeDtypeStruct((N, PW), jnp.float32),
    )


def _make_composer(B, N, D):
    info = plsc.get_sparse_core_info()
    nc, ns = info.num_cores, info.num_subcores
    nw = nc * ns
    chunk = B // nw
    n_gather = chunk // IDX_W
    W = D + D + 2

    mesh = plsc.VectorSubcoreMesh(core_axis_name="c", subcore_axis_name="s")

    @functools.partial(
        pl.kernel,
        mesh=mesh,
        compiler_params=pltpu.CompilerParams(use_tc_tiling_on_sc=False,
                                             needs_layout_passes=False),
        out_type=jax.ShapeDtypeStruct((B, W), jnp.float32),
        scratch_types=[
            pltpu.VMEM((chunk, 3), jnp.int32),
            pltpu.VMEM((n_gather, IDX_W), jnp.int32),
            pltpu.VMEM((n_gather, IDX_W), jnp.int32),
            pltpu.VMEM((2, IDX_W, PW), jnp.float32),
            pltpu.VMEM((2, IDX_W, PW), jnp.float32),
            pltpu.VMEM((chunk, W), jnp.float32),
            pltpu.SemaphoreType.DMA,
            pltpu.SemaphoreType.DMA,
            pltpu.SemaphoreType.DMA,
            pltpu.SemaphoreType.DMA,
        ],
    )
    def composer(batch_hbm, s_hbm, out_hbm, batch_v, idx_obj, idx_nb,
                 st_obj, st_nb, rows_v, sem_o0, sem_o1, sem_n0, sem_n1):
        sem_o = (sem_o0, sem_o1)
        sem_n = (sem_n0, sem_n1)
        cid = lax.axis_index("c")
        sid = lax.axis_index("s")
        wid = cid * ns + sid
        base = wid * chunk

        pltpu.sync_copy(batch_hbm.at[pl.ds(base, chunk)], batch_v)

        iota = lax.iota(jnp.int32, L)
        c0 = jnp.zeros((L,), jnp.int32)
        c1 = jnp.full((L,), 1, jnp.int32)
        c2 = jnp.full((L,), 2, jnp.int32)
        ct = jnp.full((L,), 2 * D, jnp.int32)
        co = jnp.full((L,), 2 * D + 1, jnp.int32)

        for j in range(chunk // L):
            r = iota + j * L
            o = plsc.load_gather(batch_v, [r, c0])
            n = plsc.load_gather(batch_v, [r, c1])
            t = plsc.load_gather(batch_v, [r, c2])
            idx_obj[j // (IDX_W // L), pl.ds((j % (IDX_W // L)) * L, L)] = o
            idx_nb[j // (IDX_W // L), pl.ds((j % (IDX_W // L)) * L, L)] = n
            plsc.store_scatter(rows_v, [r, ct], t.astype(jnp.float32))
            plsc.store_scatter(rows_v, [r, co], o.astype(jnp.float32))

        def fire(g):
            ho = pltpu.async_copy(s_hbm.at[idx_obj.at[g]],
                                  st_obj.at[g % 2], sem_o[g % 2])
            hn = pltpu.async_copy(s_hbm.at[idx_nb.at[g]],
                                  st_nb.at[g % 2], sem_n[g % 2])
            return ho, hn

        pend = fire(0)
        for g in range(n_gather):
            ho, hn = pend
            if g + 1 < n_gather:
                nxt = fire(g + 1)
            ho.wait()
            hn.wait()
            b = g % 2
            grow = g * IDX_W

            def body(r, carry):
                for c in range(D // L):
                    rows_v[grow + r, pl.ds(c * L, L)] = (
                        st_obj[b, r, pl.ds(c * L, L)])
                for c in range(D // L):
                    rows_v[grow + r, pl.ds(D + c * L, L)] = (
                        st_nb[b, r, pl.ds(c * L, L)])
                return carry

            lax.fori_loop(0, IDX_W, body, 0)
            if g + 1 < n_gather:
                pend = nxt

        pltpu.sync_copy(rows_v, out_hbm.at[pl.ds(base, chunk)])

    return composer


def kernel(batch, S):
    B = batch.shape[0]
    N, D = S.shape
    s_padded = _make_transpose_pad(N, D)(S.T)
    return _make_composer(B, N, D)(batch, s_padded)


# speedup vs baseline: 2.0245x; 1.6378x over previous
import functools

import jax
import jax.numpy as jnp
from jax import lax
from jax.experimental import pallas as pl
from jax.experimental.pallas import tpu as pltpu
from jax.experimental.pallas import tpu_sc as plsc

L = 16
IDX_W = 64
PW = 128


def _make_transpose_pad(N, D):
    BI = 16384
    grid = (N + BI - 1) // BI

    def body(s_t_ref, out_ref):
        x = s_t_ref[...]
        eye = jnp.eye(128, dtype=jnp.float32)
        for c in range(BI // 128):
            xc = x[:, c * 128:(c + 1) * 128]
            xt = jax.lax.dot_general(
                eye, xc, dimension_numbers=(((1,), (1,)), ((), ())),
                preferred_element_type=jnp.float32)
            out_ref[pl.ds(c * 128, 128), 0:D] = xt

    return pl.pallas_call(
        body,
        grid=(grid,),
        in_specs=[pl.BlockSpec((D, BI), lambda j: (0, j))],
        out_specs=pl.BlockSpec((BI, PW), lambda j: (j, 0)),
        out_shape=jax.ShapeDtypeStruct((N, PW), jnp.float32),
    )


def _make_composer(B, N, D):
    info = plsc.get_sparse_core_info()
    nc, ns = info.num_cores, info.num_subcores
    nw = nc * ns
    chunk = B // nw
    n_gather = chunk // IDX_W
    W = D + D + 2

    mesh = plsc.VectorSubcoreMesh(core_axis_name="c", subcore_axis_name="s")

    @functools.partial(
        pl.kernel,
        mesh=mesh,
        compiler_params=pltpu.CompilerParams(use_tc_tiling_on_sc=False,
                                             needs_layout_passes=False),
        out_type=jax.ShapeDtypeStruct((B, W), jnp.float32),
        scratch_types=[
            pltpu.VMEM((chunk, 3), jnp.int32),
            pltpu.VMEM((n_gather, IDX_W), jnp.int32),
            pltpu.VMEM((n_gather, IDX_W), jnp.int32),
            pltpu.VMEM((2, IDX_W, PW), jnp.float32),
            pltpu.VMEM((2, IDX_W, PW), jnp.float32),
            pltpu.VMEM((chunk, W), jnp.float32),
            pltpu.SemaphoreType.DMA,
            pltpu.SemaphoreType.DMA,
            pltpu.SemaphoreType.DMA,
            pltpu.SemaphoreType.DMA,
        ],
    )
    def composer(batch_hbm, s_hbm, out_hbm, batch_v, idx_obj, idx_nb,
                 st_obj, st_nb, rows_v, sem_o0, sem_o1, sem_n0, sem_n1):
        sem_o = (sem_o0, sem_o1)
        sem_n = (sem_n0, sem_n1)
        cid = lax.axis_index("c")
        sid = lax.axis_index("s")
        wid = cid * ns + sid
        base = wid * chunk

        pltpu.sync_copy(batch_hbm.at[pl.ds(base, chunk)], batch_v)

        iota = lax.iota(jnp.int32, L)
        c0 = jnp.zeros((L,), jnp.int32)
        c1 = jnp.full((L,), 1, jnp.int32)
        c2 = jnp.full((L,), 2, jnp.int32)
        ct = jnp.full((L,), 2 * D, jnp.int32)
        co = jnp.full((L,), 2 * D + 1, jnp.int32)

        for j in range(chunk // L):
            r = iota + j * L
            o = plsc.load_gather(batch_v, [r, c0])
            n = plsc.load_gather(batch_v, [r, c1])
            t = plsc.load_gather(batch_v, [r, c2])
            idx_obj[j // (IDX_W // L), pl.ds((j % (IDX_W // L)) * L, L)] = o
            idx_nb[j // (IDX_W // L), pl.ds((j % (IDX_W // L)) * L, L)] = n
            plsc.store_scatter(rows_v, [r, ct], t.astype(jnp.float32))
            plsc.store_scatter(rows_v, [r, co], o.astype(jnp.float32))

        def fire(g):
            ho = pltpu.async_copy(s_hbm.at[idx_obj.at[g]],
                                  st_obj.at[g % 2], sem_o[g % 2])
            hn = pltpu.async_copy(s_hbm.at[idx_nb.at[g]],
                                  st_nb.at[g % 2], sem_n[g % 2])
            return ho, hn

        pend = fire(0)
        for g in range(n_gather):
            ho, hn = pend
            if g + 1 < n_gather:
                nxt = fire(g + 1)
            ho.wait()
            hn.wait()
            b = g % 2
            grow = g * IDX_W

            def body(r, carry):
                for c in range(D // L):
                    rows_v[grow + r, pl.ds(c * L, L)] = (
                        st_obj[b, r, pl.ds(c * L, L)])
                for c in range(D // L):
                    rows_v[grow + r, pl.ds(D + c * L, L)] = (
                        st_nb[b, r, pl.ds(c * L, L)])
                return carry

            lax.fori_loop(0, IDX_W, body, 0)
            if g + 1 < n_gather:
                pend = nxt

        pltpu.sync_copy(rows_v, out_hbm.at[pl.ds(base, chunk)])

    return composer


def kernel(batch, S):
    B = batch.shape[0]
    N, D = S.shape
    s_padded = _make_transpose_pad(N, D)(S.T)
    return _make_composer(B, N, D)(batch, s_padded)


# speedup vs baseline: 2.0651x; 1.0200x over previous
import functools

import jax
import jax.numpy as jnp
from jax import lax
from jax.experimental import pallas as pl
from jax.experimental.pallas import tpu as pltpu
from jax.experimental.pallas import tpu_sc as plsc

L = 16
IDX_W = 64
PW = 128


def _make_transpose_pad(N, D):
    BI = 32768
    grid = (N + BI - 1) // BI

    def body(s_t_ref, out_ref):
        x = s_t_ref[...]
        eye = jnp.eye(128, dtype=jnp.float32)
        for c in range(BI // 128):
            xc = x[:, c * 128:(c + 1) * 128]
            xt = jax.lax.dot_general(
                eye, xc, dimension_numbers=(((1,), (1,)), ((), ())),
                preferred_element_type=jnp.float32)
            out_ref[pl.ds(c * 128, 128), 0:D] = xt

    return pl.pallas_call(
        body,
        grid=(grid,),
        in_specs=[pl.BlockSpec((D, BI), lambda j: (0, j))],
        out_specs=pl.BlockSpec((BI, PW), lambda j: (j, 0)),
        out_shape=jax.ShapeDtypeStruct((N, PW), jnp.float32),
    )


def _make_composer(B, N, D):
    info = plsc.get_sparse_core_info()
    nc, ns = info.num_cores, info.num_subcores
    nw = nc * ns
    chunk = B // nw
    n_gather = chunk // IDX_W
    W = D + D + 2

    mesh = plsc.VectorSubcoreMesh(core_axis_name="c", subcore_axis_name="s")

    @functools.partial(
        pl.kernel,
        mesh=mesh,
        compiler_params=pltpu.CompilerParams(use_tc_tiling_on_sc=False,
                                             needs_layout_passes=False),
        out_type=jax.ShapeDtypeStruct((B, W), jnp.float32),
        scratch_types=[
            pltpu.VMEM((chunk, 3), jnp.int32),
            pltpu.VMEM((n_gather, IDX_W), jnp.int32),
            pltpu.VMEM((n_gather, IDX_W), jnp.int32),
            pltpu.VMEM((2, IDX_W, PW), jnp.float32),
            pltpu.VMEM((2, IDX_W, PW), jnp.float32),
            pltpu.VMEM((chunk, W), jnp.float32),
            pltpu.SemaphoreType.DMA,
            pltpu.SemaphoreType.DMA,
            pltpu.SemaphoreType.DMA,
            pltpu.SemaphoreType.DMA,
        ],
    )
    def composer(batch_hbm, s_hbm, out_hbm, batch_v, idx_obj, idx_nb,
                 st_obj, st_nb, rows_v, sem_o0, sem_o1, sem_n0, sem_n1):
        sem_o = (sem_o0, sem_o1)
        sem_n = (sem_n0, sem_n1)
        cid = lax.axis_index("c")
        sid = lax.axis_index("s")
        wid = cid * ns + sid
        base = wid * chunk

        pltpu.sync_copy(batch_hbm.at[pl.ds(base, chunk)], batch_v)

        iota = lax.iota(jnp.int32, L)
        c0 = jnp.zeros((L,), jnp.int32)
        c1 = jnp.full((L,), 1, jnp.int32)
        c2 = jnp.full((L,), 2, jnp.int32)
        ct = jnp.full((L,), 2 * D, jnp.int32)
        co = jnp.full((L,), 2 * D + 1, jnp.int32)

        for j in range(chunk // L):
            r = iota + j * L
            o = plsc.load_gather(batch_v, [r, c0])
            n = plsc.load_gather(batch_v, [r, c1])
            t = plsc.load_gather(batch_v, [r, c2])
            idx_obj[j // (IDX_W // L), pl.ds((j % (IDX_W // L)) * L, L)] = o
            idx_nb[j // (IDX_W // L), pl.ds((j % (IDX_W // L)) * L, L)] = n
            plsc.store_scatter(rows_v, [r, ct], t.astype(jnp.float32))
            plsc.store_scatter(rows_v, [r, co], o.astype(jnp.float32))

        def fire(g):
            ho = pltpu.async_copy(s_hbm.at[idx_obj.at[g]],
                                  st_obj.at[g % 2], sem_o[g % 2])
            hn = pltpu.async_copy(s_hbm.at[idx_nb.at[g]],
                                  st_nb.at[g % 2], sem_n[g % 2])
            return ho, hn

        pend = fire(0)
        for g in range(n_gather):
            ho, hn = pend
            if g + 1 < n_gather:
                nxt = fire(g + 1)
            ho.wait()
            hn.wait()
            b = g % 2
            grow = g * IDX_W

            def body(r, carry):
                for c in range(D // L):
                    rows_v[grow + r, pl.ds(c * L, L)] = (
                        st_obj[b, r, pl.ds(c * L, L)])
                for c in range(D // L):
                    rows_v[grow + r, pl.ds(D + c * L, L)] = (
                        st_nb[b, r, pl.ds(c * L, L)])
                return carry

            lax.fori_loop(0, IDX_W, body, 0)
            if g + 1 < n_gather:
                pend = nxt

        pltpu.sync_copy(rows_v, out_hbm.at[pl.ds(base, chunk)])

    return composer


def kernel(batch, S):
    B = batch.shape[0]
    N, D = S.shape
    s_padded = _make_transpose_pad(N, D)(S.T)
    return _make_composer(B, N, D)(batch, s_padded)


# speedup vs baseline: 2.3668x; 1.1461x over previous
import functools

import jax
import jax.numpy as jnp
from jax import lax
from jax.experimental import pallas as pl
from jax.experimental.pallas import tpu as pltpu
from jax.experimental.pallas import tpu_sc as plsc

L = 16
IDX_W = 128


def _make_transpose_pair(N, D):
    BI = 16384
    grid = (N + BI - 1) // BI

    def body(s_t_ref, out_ref):
        x = s_t_ref[...]
        k2 = lax.broadcasted_iota(jnp.int32, (128, 256), 0)
        j2 = lax.broadcasted_iota(jnp.int32, (128, 256), 1)
        pe = (j2 == 2 * k2).astype(jnp.float32)
        po = (j2 == 2 * k2 + 1).astype(jnp.float32)
        for c in range(BI // 256):
            xc = x[:, c * 256:(c + 1) * 256]
            xte = jax.lax.dot_general(
                pe, xc, dimension_numbers=(((1,), (1,)), ((), ())),
                preferred_element_type=jnp.float32)
            xto = jax.lax.dot_general(
                po, xc, dimension_numbers=(((1,), (1,)), ((), ())),
                preferred_element_type=jnp.float32)
            out_ref[pl.ds(c * 128, 128), 0:D] = xte
            out_ref[pl.ds(c * 128, 128), D:2 * D] = xto

    return pl.pallas_call(
        body,
        grid=(grid,),
        in_specs=[pl.BlockSpec((D, BI), lambda j: (0, j))],
        out_specs=pl.BlockSpec((BI // 2, 2 * D), lambda j: (j, 0)),
        out_shape=jax.ShapeDtypeStruct((N // 2, 2 * D), jnp.float32),
    )


def _make_composer(B, N, D):
    info = plsc.get_sparse_core_info()
    nc, ns = info.num_cores, info.num_subcores
    nw = nc * ns
    chunk = B // nw
    n_gather = chunk // IDX_W
    W = D + D + 2

    mesh = plsc.VectorSubcoreMesh(core_axis_name="c", subcore_axis_name="s")

    @functools.partial(
        pl.kernel,
        mesh=mesh,
        compiler_params=pltpu.CompilerParams(use_tc_tiling_on_sc=False,
                                             needs_layout_passes=False),
        out_type=jax.ShapeDtypeStruct((B, W), jnp.float32),
        scratch_types=[
            pltpu.VMEM((chunk, 3), jnp.int32),
            pltpu.VMEM((n_gather, IDX_W), jnp.int32),
            pltpu.VMEM((n_gather, IDX_W), jnp.int32),
            pltpu.VMEM((2, IDX_W, D), jnp.float32),
            pltpu.VMEM((2, IDX_W, D), jnp.float32),
            pltpu.VMEM((chunk, W), jnp.float32),
            pltpu.SemaphoreType.DMA,
            pltpu.SemaphoreType.DMA,
            pltpu.SemaphoreType.DMA,
            pltpu.SemaphoreType.DMA,
        ],
    )
    def composer(batch_hbm, s_hbm, out_hbm, batch_v, idx_obj, idx_nb,
                 st_obj, st_nb, rows_v, sem_o0, sem_o1, sem_n0, sem_n1):
        sem_o = (sem_o0, sem_o1)
        sem_n = (sem_n0, sem_n1)
        cid = lax.axis_index("c")
        sid = lax.axis_index("s")
        wid = cid * ns + sid
        base = wid * chunk

        pltpu.sync_copy(batch_hbm.at[pl.ds(base, chunk)], batch_v)

        iota = lax.iota(jnp.int32, L)
        c0 = jnp.zeros((L,), jnp.int32)
        c1 = jnp.full((L,), 1, jnp.int32)
        c2 = jnp.full((L,), 2, jnp.int32)
        ct = jnp.full((L,), 2 * D, jnp.int32)
        co = jnp.full((L,), 2 * D + 1, jnp.int32)

        for j in range(chunk // L):
            r = iota + j * L
            o = plsc.load_gather(batch_v, [r, c0])
            n = plsc.load_gather(batch_v, [r, c1])
            t = plsc.load_gather(batch_v, [r, c2])
            idx_obj[j // (IDX_W // L), pl.ds((j % (IDX_W // L)) * L, L)] = o
            idx_nb[j // (IDX_W // L), pl.ds((j % (IDX_W // L)) * L, L)] = n
            plsc.store_scatter(rows_v, [r, ct], t.astype(jnp.float32))
            plsc.store_scatter(rows_v, [r, co], o.astype(jnp.float32))

        def fire(g):
            ho = pltpu.async_copy(s_hbm.at[idx_obj.at[g]],
                                  st_obj.at[g % 2], sem_o[g % 2])
            hn = pltpu.async_copy(s_hbm.at[idx_nb.at[g]],
                                  st_nb.at[g % 2], sem_n[g % 2])
            return ho, hn

        pend = fire(0)
        for g in range(n_gather):
            ho, hn = pend
            if g + 1 < n_gather:
                nxt = fire(g + 1)
            ho.wait()
            hn.wait()
            b = g % 2
            grow = g * IDX_W

            def body(r, carry):
                for c in range(D // L):
                    rows_v[grow + r, pl.ds(c * L, L)] = (
                        st_obj[b, r, pl.ds(c * L, L)])
                for c in range(D // L):
                    rows_v[grow + r, pl.ds(D + c * L, L)] = (
                        st_nb[b, r, pl.ds(c * L, L)])
                return carry

            lax.fori_loop(0, IDX_W, body, 0)
            if g + 1 < n_gather:
                pend = nxt

        pltpu.sync_copy(rows_v, out_hbm.at[pl.ds(base, chunk)])

    return composer


def kernel(batch, S):
    B = batch.shape[0]
    N, D = S.shape
    s_pairs = _make_transpose_pair(N, D)(S.T)
    s_lin = s_pairs.reshape(N, D)
    return _make_composer(B, N, D)(batch, s_lin)


# speedup vs baseline: 2.5010x; 1.0567x over previous
import functools

import jax
import jax.numpy as jnp
from jax import lax
from jax.experimental import pallas as pl
from jax.experimental.pallas import tpu as pltpu
from jax.experimental.pallas import tpu_sc as plsc

L = 16
IDX_W = 128


def _make_transpose_pair(N, D):
    BI = 24576
    grid = (N + BI - 1) // BI

    def body(s_t_ref, out_ref):
        x = s_t_ref[...]
        k2 = lax.broadcasted_iota(jnp.int32, (128, 256), 0)
        j2 = lax.broadcasted_iota(jnp.int32, (128, 256), 1)
        pe = (j2 == 2 * k2).astype(jnp.float32)
        po = (j2 == 2 * k2 + 1).astype(jnp.float32)
        for c in range(BI // 256):
            xc = x[:, c * 256:(c + 1) * 256]
            xte = jax.lax.dot_general(
                pe, xc, dimension_numbers=(((1,), (1,)), ((), ())),
                preferred_element_type=jnp.float32)
            xto = jax.lax.dot_general(
                po, xc, dimension_numbers=(((1,), (1,)), ((), ())),
                preferred_element_type=jnp.float32)
            out_ref[pl.ds(c * 128, 128), 0:D] = xte
            out_ref[pl.ds(c * 128, 128), D:2 * D] = xto

    return pl.pallas_call(
        body,
        grid=(grid,),
        in_specs=[pl.BlockSpec((D, BI), lambda j: (0, j))],
        out_specs=pl.BlockSpec((BI // 2, 2 * D), lambda j: (j, 0)),
        out_shape=jax.ShapeDtypeStruct((N // 2, 2 * D), jnp.float32),
    )


def _make_composer(B, N, D):
    info = plsc.get_sparse_core_info()
    nc, ns = info.num_cores, info.num_subcores
    nw = nc * ns
    chunk = B // nw
    n_gather = chunk // IDX_W
    W = D + D + 2

    mesh = plsc.VectorSubcoreMesh(core_axis_name="c", subcore_axis_name="s")

    @functools.partial(
        pl.kernel,
        mesh=mesh,
        compiler_params=pltpu.CompilerParams(use_tc_tiling_on_sc=False,
                                             needs_layout_passes=False),
        out_type=jax.ShapeDtypeStruct((B, W), jnp.float32),
        scratch_types=[
            pltpu.VMEM((chunk, 3), jnp.int32),
            pltpu.VMEM((n_gather, IDX_W), jnp.int32),
            pltpu.VMEM((n_gather, IDX_W), jnp.int32),
            pltpu.VMEM((2, IDX_W, D), jnp.float32),
            pltpu.VMEM((2, IDX_W, D), jnp.float32),
            pltpu.VMEM((chunk, W), jnp.float32),
            pltpu.SemaphoreType.DMA,
            pltpu.SemaphoreType.DMA,
            pltpu.SemaphoreType.DMA,
            pltpu.SemaphoreType.DMA,
        ],
    )
    def composer(batch_hbm, s_hbm, out_hbm, batch_v, idx_obj, idx_nb,
                 st_obj, st_nb, rows_v, sem_o0, sem_o1, sem_n0, sem_n1):
        sem_o = (sem_o0, sem_o1)
        sem_n = (sem_n0, sem_n1)
        cid = lax.axis_index("c")
        sid = lax.axis_index("s")
        wid = cid * ns + sid
        base = wid * chunk

        pltpu.sync_copy(batch_hbm.at[pl.ds(base, chunk)], batch_v)

        iota = lax.iota(jnp.int32, L)
        c0 = jnp.zeros((L,), jnp.int32)
        c1 = jnp.full((L,), 1, jnp.int32)
        c2 = jnp.full((L,), 2, jnp.int32)
        ct = jnp.full((L,), 2 * D, jnp.int32)
        co = jnp.full((L,), 2 * D + 1, jnp.int32)

        for j in range(chunk // L):
            r = iota + j * L
            o = plsc.load_gather(batch_v, [r, c0])
            n = plsc.load_gather(batch_v, [r, c1])
            t = plsc.load_gather(batch_v, [r, c2])
            idx_obj[j // (IDX_W // L), pl.ds((j % (IDX_W // L)) * L, L)] = o
            idx_nb[j // (IDX_W // L), pl.ds((j % (IDX_W // L)) * L, L)] = n
            plsc.store_scatter(rows_v, [r, ct], t.astype(jnp.float32))
            plsc.store_scatter(rows_v, [r, co], o.astype(jnp.float32))

        def fire(g):
            ho = pltpu.async_copy(s_hbm.at[idx_obj.at[g]],
                                  st_obj.at[g % 2], sem_o[g % 2])
            hn = pltpu.async_copy(s_hbm.at[idx_nb.at[g]],
                                  st_nb.at[g % 2], sem_n[g % 2])
            return ho, hn

        pend = fire(0)
        for g in range(n_gather):
            ho, hn = pend
            if g + 1 < n_gather:
                nxt = fire(g + 1)
            ho.wait()
            hn.wait()
            b = g % 2
            grow = g * IDX_W

            def body(r, carry):
                for c in range(D // L):
                    rows_v[grow + r, pl.ds(c * L, L)] = (
                        st_obj[b, r, pl.ds(c * L, L)])
                for c in range(D // L):
                    rows_v[grow + r, pl.ds(D + c * L, L)] = (
                        st_nb[b, r, pl.ds(c * L, L)])
                return carry

            lax.fori_loop(0, IDX_W, body, 0)
            if g + 1 < n_gather:
                pend = nxt

        pltpu.sync_copy(rows_v, out_hbm.at[pl.ds(base, chunk)])

    return composer


def kernel(batch, S):
    B = batch.shape[0]
    N, D = S.shape
    s_pairs = _make_transpose_pair(N, D)(S.T)
    s_lin = s_pairs.reshape(N, D)
    return _make_composer(B, N, D)(batch, s_lin)


# speedup vs baseline: 2.6141x; 1.0452x over previous
import functools

import jax
import jax.numpy as jnp
from jax import lax
from jax.experimental import pallas as pl
from jax.experimental.pallas import tpu as pltpu
from jax.experimental.pallas import tpu_sc as plsc

L = 16
IDX_W = 128


def _make_transpose_pair(N, D):
    BI = 24576
    grid = (N + BI - 1) // BI

    def body(s_t_ref, out_ref):
        x = s_t_ref[...]
        k2 = lax.broadcasted_iota(jnp.int32, (128, 256), 0)
        j2 = lax.broadcasted_iota(jnp.int32, (128, 256), 1)
        pe = (j2 == 2 * k2).astype(jnp.float32)
        po = (j2 == 2 * k2 + 1).astype(jnp.float32)
        for c in range(BI // 256):
            xc = x[:, c * 256:(c + 1) * 256]
            xte = jax.lax.dot_general(
                pe, xc, dimension_numbers=(((1,), (1,)), ((), ())),
                preferred_element_type=jnp.float32)
            xto = jax.lax.dot_general(
                po, xc, dimension_numbers=(((1,), (1,)), ((), ())),
                preferred_element_type=jnp.float32)
            out_ref[pl.ds(c * 128, 128), 0:D] = xte
            out_ref[pl.ds(c * 128, 128), D:2 * D] = xto

    return pl.pallas_call(
        body,
        grid=(grid,),
        in_specs=[pl.BlockSpec((D, BI), lambda j: (0, j))],
        out_specs=pl.BlockSpec((BI // 2, 2 * D), lambda j: (j, 0)),
        out_shape=jax.ShapeDtypeStruct((N // 2, 2 * D), jnp.float32),
    )


def _make_composer(B, N, D):
    info = plsc.get_sparse_core_info()
    nc, ns = info.num_cores, info.num_subcores
    nw = nc * ns
    chunk = B // nw
    n_gather = chunk // IDX_W
    W = D + D + 2

    mesh = plsc.VectorSubcoreMesh(core_axis_name="c", subcore_axis_name="s")

    @functools.partial(
        pl.kernel,
        mesh=mesh,
        compiler_params=pltpu.CompilerParams(use_tc_tiling_on_sc=False,
                                             needs_layout_passes=False),
        out_type=jax.ShapeDtypeStruct((B, W), jnp.float32),
        scratch_types=[
            pltpu.VMEM((n_gather, IDX_W), jnp.int32),
            pltpu.VMEM((n_gather, IDX_W), jnp.int32),
            pltpu.VMEM((n_gather, IDX_W), jnp.int32),
            pltpu.VMEM((2, IDX_W, D), jnp.float32),
            pltpu.VMEM((2, IDX_W, D), jnp.float32),
            pltpu.VMEM((chunk, W), jnp.float32),
            pltpu.SemaphoreType.DMA,
            pltpu.SemaphoreType.DMA,
            pltpu.SemaphoreType.DMA,
            pltpu.SemaphoreType.DMA,
        ],
    )
    def composer(batch_hbm, s_hbm, out_hbm, idx_obj, idx_nb, tv,
                 st_obj, st_nb, rows_v, sem_o0, sem_o1, sem_n0, sem_n1):
        sem_o = (sem_o0, sem_o1)
        sem_n = (sem_n0, sem_n1)
        cid = lax.axis_index("c")
        sid = lax.axis_index("s")
        wid = cid * ns + sid
        base = wid * chunk

        for g in range(n_gather):
            blk = pl.ds(base + g * IDX_W, IDX_W)
            pltpu.sync_copy(batch_hbm.at[0, blk], idx_obj.at[g])
            pltpu.sync_copy(batch_hbm.at[1, blk], idx_nb.at[g])
            pltpu.sync_copy(batch_hbm.at[2, blk], tv.at[g])

        iota = lax.iota(jnp.int32, L)
        ct = jnp.full((L,), 2 * D, jnp.int32)
        co = jnp.full((L,), 2 * D + 1, jnp.int32)

        for j in range(chunk // L):
            r = iota + j * L
            g, w = j // (IDX_W // L), (j % (IDX_W // L)) * L
            o = idx_obj[g, pl.ds(w, L)]
            t = tv[g, pl.ds(w, L)]
            plsc.store_scatter(rows_v, [r, ct], t.astype(jnp.float32))
            plsc.store_scatter(rows_v, [r, co], o.astype(jnp.float32))

        def fire(g):
            ho = pltpu.async_copy(s_hbm.at[idx_obj.at[g]],
                                  st_obj.at[g % 2], sem_o[g % 2])
            hn = pltpu.async_copy(s_hbm.at[idx_nb.at[g]],
                                  st_nb.at[g % 2], sem_n[g % 2])
            return ho, hn

        pend = fire(0)
        for g in range(n_gather):
            ho, hn = pend
            if g + 1 < n_gather:
                nxt = fire(g + 1)
            ho.wait()
            hn.wait()
            b = g % 2
            grow = g * IDX_W

            def body(r, carry):
                for c in range(D // L):
                    rows_v[grow + r, pl.ds(c * L, L)] = (
                        st_obj[b, r, pl.ds(c * L, L)])
                for c in range(D // L):
                    rows_v[grow + r, pl.ds(D + c * L, L)] = (
                        st_nb[b, r, pl.ds(c * L, L)])
                return carry

            lax.fori_loop(0, IDX_W, body, 0)
            if g + 1 < n_gather:
                pend = nxt

        pltpu.sync_copy(rows_v, out_hbm.at[pl.ds(base, chunk)])

    return composer


def kernel(batch, S):
    B = batch.shape[0]
    N, D = S.shape
    s_pairs = _make_transpose_pair(N, D)(S.T)
    s_lin = s_pairs.reshape(N, D)
    return _make_composer(B, N, D)(batch.T, s_lin)


# speedup vs baseline: 2.6462x; 1.0123x over previous
import functools

import jax
import jax.numpy as jnp
from jax import lax
from jax.experimental import pallas as pl
from jax.experimental.pallas import tpu as pltpu
from jax.experimental.pallas import tpu_sc as plsc

L = 16
IDX_W = 128


def _make_transpose_pair(N, D):
    BI = 28672
    grid = (N + BI - 1) // BI

    def body(s_t_ref, out_ref):
        x = s_t_ref[...]
        k2 = lax.broadcasted_iota(jnp.int32, (128, 256), 0)
        j2 = lax.broadcasted_iota(jnp.int32, (128, 256), 1)
        pe = (j2 == 2 * k2).astype(jnp.float32)
        po = (j2 == 2 * k2 + 1).astype(jnp.float32)
        for c in range(BI // 256):
            xc = x[:, c * 256:(c + 1) * 256]
            xte = jax.lax.dot_general(
                pe, xc, dimension_numbers=(((1,), (1,)), ((), ())),
                preferred_element_type=jnp.float32)
            xto = jax.lax.dot_general(
                po, xc, dimension_numbers=(((1,), (1,)), ((), ())),
                preferred_element_type=jnp.float32)
            out_ref[pl.ds(c * 128, 128), 0:D] = xte
            out_ref[pl.ds(c * 128, 128), D:2 * D] = xto

    return pl.pallas_call(
        body,
        grid=(grid,),
        in_specs=[pl.BlockSpec((D, BI), lambda j: (0, j))],
        out_specs=pl.BlockSpec((BI // 2, 2 * D), lambda j: (j, 0)),
        out_shape=jax.ShapeDtypeStruct((N // 2, 2 * D), jnp.float32),
    )


def _make_composer(B, N, D):
    info = plsc.get_sparse_core_info()
    nc, ns = info.num_cores, info.num_subcores
    nw = nc * ns
    chunk = B // nw
    n_gather = chunk // IDX_W
    W = D + D + 2

    mesh = plsc.VectorSubcoreMesh(core_axis_name="c", subcore_axis_name="s")

    @functools.partial(
        pl.kernel,
        mesh=mesh,
        compiler_params=pltpu.CompilerParams(use_tc_tiling_on_sc=False,
                                             needs_layout_passes=False),
        out_type=jax.ShapeDtypeStruct((B, W), jnp.float32),
        scratch_types=[
            pltpu.VMEM((n_gather, IDX_W), jnp.int32),
            pltpu.VMEM((n_gather, IDX_W), jnp.int32),
            pltpu.VMEM((n_gather, IDX_W), jnp.int32),
            pltpu.VMEM((2, IDX_W, D), jnp.float32),
            pltpu.VMEM((2, IDX_W, D), jnp.float32),
            pltpu.VMEM((chunk, W), jnp.float32),
            pltpu.SemaphoreType.DMA,
            pltpu.SemaphoreType.DMA,
            pltpu.SemaphoreType.DMA,
            pltpu.SemaphoreType.DMA,
        ],
    )
    def composer(batch_hbm, s_hbm, out_hbm, idx_obj, idx_nb, tv,
                 st_obj, st_nb, rows_v, sem_o0, sem_o1, sem_n0, sem_n1):
        sem_o = (sem_o0, sem_o1)
        sem_n = (sem_n0, sem_n1)
        cid = lax.axis_index("c")
        sid = lax.axis_index("s")
        wid = cid * ns + sid
        base = wid * chunk

        for g in range(n_gather):
            blk = pl.ds(base + g * IDX_W, IDX_W)
            pltpu.sync_copy(batch_hbm.at[0, blk], idx_obj.at[g])
            pltpu.sync_copy(batch_hbm.at[1, blk], idx_nb.at[g])
            pltpu.sync_copy(batch_hbm.at[2, blk], tv.at[g])

        iota = lax.iota(jnp.int32, L)
        ct = jnp.full((L,), 2 * D, jnp.int32)
        co = jnp.full((L,), 2 * D + 1, jnp.int32)

        for j in range(chunk // L):
            r = iota + j * L
            g, w = j // (IDX_W // L), (j % (IDX_W // L)) * L
            o = idx_obj[g, pl.ds(w, L)]
            t = tv[g, pl.ds(w, L)]
            plsc.store_scatter(rows_v, [r, ct], t.astype(jnp.float32))
            plsc.store_scatter(rows_v, [r, co], o.astype(jnp.float32))

        def fire(g):
            ho = pltpu.async_copy(s_hbm.at[idx_obj.at[g]],
                                  st_obj.at[g % 2], sem_o[g % 2])
            hn = pltpu.async_copy(s_hbm.at[idx_nb.at[g]],
                                  st_nb.at[g % 2], sem_n[g % 2])
            return ho, hn

        pend = fire(0)
        for g in range(n_gather):
            ho, hn = pend
            if g + 1 < n_gather:
                nxt = fire(g + 1)
            ho.wait()
            hn.wait()
            b = g % 2
            grow = g * IDX_W

            def body(r, carry):
                for c in range(D // L):
                    rows_v[grow + r, pl.ds(c * L, L)] = (
                        st_obj[b, r, pl.ds(c * L, L)])
                for c in range(D // L):
                    rows_v[grow + r, pl.ds(D + c * L, L)] = (
                        st_nb[b, r, pl.ds(c * L, L)])
                return carry

            lax.fori_loop(0, IDX_W, body, 0)
            if g + 1 < n_gather:
                pend = nxt

        pltpu.sync_copy(rows_v, out_hbm.at[pl.ds(base, chunk)])

    return composer


def kernel(batch, S):
    B = batch.shape[0]
    N, D = S.shape
    s_pairs = _make_transpose_pair(N, D)(S.T)
    s_lin = s_pairs.reshape(N, D)
    return _make_composer(B, N, D)(batch.T, s_lin)


# speedup vs baseline: 2.6586x; 1.0047x over previous
import functools

import jax
import jax.numpy as jnp
from jax import lax
from jax.experimental import pallas as pl
from jax.experimental.pallas import tpu as pltpu
from jax.experimental.pallas import tpu_sc as plsc

L = 16
IDX_W = 128


def _make_transpose_pair(N, D):
    BI = 30720
    grid = (N + BI - 1) // BI

    def body(s_t_ref, out_ref):
        x = s_t_ref[...]
        k2 = lax.broadcasted_iota(jnp.int32, (128, 256), 0)
        j2 = lax.broadcasted_iota(jnp.int32, (128, 256), 1)
        pe = (j2 == 2 * k2).astype(jnp.float32)
        po = (j2 == 2 * k2 + 1).astype(jnp.float32)
        for c in range(BI // 256):
            xc = x[:, c * 256:(c + 1) * 256]
            xte = jax.lax.dot_general(
                pe, xc, dimension_numbers=(((1,), (1,)), ((), ())),
                preferred_element_type=jnp.float32)
            xto = jax.lax.dot_general(
                po, xc, dimension_numbers=(((1,), (1,)), ((), ())),
                preferred_element_type=jnp.float32)
            out_ref[pl.ds(c * 128, 128), 0:D] = xte
            out_ref[pl.ds(c * 128, 128), D:2 * D] = xto

    return pl.pallas_call(
        body,
        grid=(grid,),
        in_specs=[pl.BlockSpec((D, BI), lambda j: (0, j))],
        out_specs=pl.BlockSpec((BI // 2, 2 * D), lambda j: (j, 0)),
        out_shape=jax.ShapeDtypeStruct((N // 2, 2 * D), jnp.float32),
    )


def _make_composer(B, N, D):
    info = plsc.get_sparse_core_info()
    nc, ns = info.num_cores, info.num_subcores
    nw = nc * ns
    chunk = B // nw
    n_gather = chunk // IDX_W
    W = D + D + 2

    mesh = plsc.VectorSubcoreMesh(core_axis_name="c", subcore_axis_name="s")

    @functools.partial(
        pl.kernel,
        mesh=mesh,
        compiler_params=pltpu.CompilerParams(use_tc_tiling_on_sc=False,
                                             needs_layout_passes=False),
        out_type=jax.ShapeDtypeStruct((B, W), jnp.float32),
        scratch_types=[
            pltpu.VMEM((n_gather, IDX_W), jnp.int32),
            pltpu.VMEM((n_gather, IDX_W), jnp.int32),
            pltpu.VMEM((n_gather, IDX_W), jnp.int32),
            pltpu.VMEM((2, IDX_W, D), jnp.float32),
            pltpu.VMEM((2, IDX_W, D), jnp.float32),
            pltpu.VMEM((chunk, W), jnp.float32),
            pltpu.SemaphoreType.DMA,
            pltpu.SemaphoreType.DMA,
            pltpu.SemaphoreType.DMA,
            pltpu.SemaphoreType.DMA,
        ],
    )
    def composer(batch_hbm, s_hbm, out_hbm, idx_obj, idx_nb, tv,
                 st_obj, st_nb, rows_v, sem_o0, sem_o1, sem_n0, sem_n1):
        sem_o = (sem_o0, sem_o1)
        sem_n = (sem_n0, sem_n1)
        cid = lax.axis_index("c")
        sid = lax.axis_index("s")
        wid = cid * ns + sid
        base = wid * chunk

        for g in range(n_gather):
            blk = pl.ds(base + g * IDX_W, IDX_W)
            pltpu.sync_copy(batch_hbm.at[0, blk], idx_obj.at[g])
            pltpu.sync_copy(batch_hbm.at[1, blk], idx_nb.at[g])
            pltpu.sync_copy(batch_hbm.at[2, blk], tv.at[g])

        iota = lax.iota(jnp.int32, L)
        ct = jnp.full((L,), 2 * D, jnp.int32)
        co = jnp.full((L,), 2 * D + 1, jnp.int32)

        for j in range(chunk // L):
            r = iota + j * L
            g, w = j // (IDX_W // L), (j % (IDX_W // L)) * L
            o = idx_obj[g, pl.ds(w, L)]
            t = tv[g, pl.ds(w, L)]
            plsc.store_scatter(rows_v, [r, ct], t.astype(jnp.float32))
            plsc.store_scatter(rows_v, [r, co], o.astype(jnp.float32))

        def fire(g):
            ho = pltpu.async_copy(s_hbm.at[idx_obj.at[g]],
                                  st_obj.at[g % 2], sem_o[g % 2])
            hn = pltpu.async_copy(s_hbm.at[idx_nb.at[g]],
                                  st_nb.at[g % 2], sem_n[g % 2])
            return ho, hn

        pend = fire(0)
        for g in range(n_gather):
            ho, hn = pend
            if g + 1 < n_gather:
                nxt = fire(g + 1)
            ho.wait()
            hn.wait()
            b = g % 2
            grow = g * IDX_W

            def body(r, carry):
                for c in range(D // L):
                    rows_v[grow + r, pl.ds(c * L, L)] = (
                        st_obj[b, r, pl.ds(c * L, L)])
                for c in range(D // L):
                    rows_v[grow + r, pl.ds(D + c * L, L)] = (
                        st_nb[b, r, pl.ds(c * L, L)])
                return carry

            lax.fori_loop(0, IDX_W, body, 0)
            if g + 1 < n_gather:
                pend = nxt

        pltpu.sync_copy(rows_v, out_hbm.at[pl.ds(base, chunk)])

    return composer


def kernel(batch, S):
    B = batch.shape[0]
    N, D = S.shape
    s_pairs = _make_transpose_pair(N, D)(S.T)
    s_lin = s_pairs.reshape(N, D)
    return _make_composer(B, N, D)(batch.T, s_lin)
